# baseline (speedup 1.0000x reference)
import jax
import jax.numpy as jnp
from jax.experimental import pallas as pl

S_BLK = 2048


def _geno_block(xt_ref, a_ref, p_ref, o_ref):
    y = jax.lax.dot_general(
        xt_ref[0], a_ref[...],
        dimension_numbers=(((0,), (0,)), ((), ())),
        preferred_element_type=jnp.float32,
    )
    o_ref[0] = y + p_ref[...]


@jax.jit
def kernel(x, allele_embedding, position_embedding):
    B, S, N = x.shape
    D = allele_embedding.shape[1]
    xt = x.transpose(0, 2, 1)
    grid = (S // S_BLK, B)
    out = pl.pallas_call(
        _geno_block,
        grid=grid,
        in_specs=[
            pl.BlockSpec((1, N, S_BLK), lambda i, b: (b, 0, i)),
            pl.BlockSpec((N, D), lambda i, b: (0, 0)),
            pl.BlockSpec((S_BLK, D), lambda i, b: (i, 0)),
        ],
        out_specs=pl.BlockSpec((1, S_BLK, D), lambda i, b: (b, i, 0)),
        out_shape=jax.ShapeDtypeStruct((B, S, D), jnp.float32),
    )(xt, allele_embedding, position_embedding)
    return out
